# baseline (speedup 1.0000x reference)
import jax
import jax.numpy as jnp
from jax import lax
from jax.experimental import pallas as pl
from jax.experimental.pallas import tpu as pltpu
from jax.experimental.pallas import tpu_sc as plsc

_NC = 19
_NCP = 24
_T = 8192

_NW = 32
_CH = 1536


def _sc_route(preds):
    b, nc, hw = preds.shape
    per_w = hw // _NW
    nch = per_w // _CH
    mesh = plsc.VectorSubcoreMesh(core_axis_name="c", subcore_axis_name="s")

    def body(preds_hbm, m_hbm, k_hbm, rows_v, m_buf, k_buf, sem):
        wid = lax.axis_index("s") * 2 + lax.axis_index("c")
        base = wid * per_w
        for bi in range(b):
            for ci in range(nch):
                start = base + ci * _CH
                cps = [
                    pltpu.async_copy(
                        preds_hbm.at[pl.ds((bi * nc + cls) * hw + start, _CH)],
                        rows_v.at[pl.ds(cls * _CH, _CH)], sem)
                    for cls in range(nc)
                ]
                for cp in cps:
                    cp.wait()

                def step(i, carry):
                    off = i * 16
                    mv = rows_v[pl.ds(off, 16)]
                    kv = jnp.zeros((16,), jnp.int32)
                    for cls in range(1, nc):
                        v = rows_v[pl.ds(cls * _CH + off, 16)]
                        gt = v > mv
                        mv = jnp.where(gt, v, mv)
                        kv = jnp.where(gt, cls, kv)
                    m_buf[pl.ds(off, 16)] = mv
                    k_buf[pl.ds(off, 16)] = kv
                    return carry

                lax.fori_loop(0, _CH // 16, step, 0)
                pltpu.sync_copy(m_buf, m_hbm.at[pl.ds(bi * hw + start, _CH)])
                pltpu.sync_copy(k_buf, k_hbm.at[pl.ds(bi * hw + start, _CH)])

    f = pl.kernel(
        body,
        mesh=mesh,
        out_type=[
            jax.ShapeDtypeStruct((b * hw,), jnp.float32),
            jax.ShapeDtypeStruct((b * hw,), jnp.int32),
        ],
        scratch_types=[
            pltpu.VMEM((nc * _CH,), jnp.float32),
            pltpu.VMEM((_CH,), jnp.float32),
            pltpu.VMEM((_CH,), jnp.int32),
            pltpu.SemaphoreType.DMA,
        ],
    )
    return f(preds.reshape(-1))


def _tc_body(x_ref, m_ref, k_ref, w1_ref, b1_ref, w2_ref, b2_ref, out_ref,
             nw_ref, at_ref, mr_ref, d_ref):
    j = pl.program_id(1)
    nblk = pl.num_programs(1)
    c = x_ref.shape[1]

    @pl.when(j == 0)
    def _init():
        nw_ref[...] = jnp.zeros_like(nw_ref)
        at_ref[...] = jnp.zeros_like(at_ref)
        d_ref[...] = jnp.zeros_like(d_ref)
        mr_ref[...] = jnp.full_like(mr_ref, -jnp.inf)

    x = x_ref[0]
    m = m_ref[0]
    k = k_ref[0]
    rows = jax.lax.broadcasted_iota(jnp.int32, (_NCP, _T), 0)
    sel = rows == k
    maskf = sel.astype(jnp.float32)

    mb = jnp.where(sel, jnp.broadcast_to(m, (_NCP, _T)), -jnp.inf)
    pm = jnp.max(mb, axis=1, keepdims=True)
    m_old = mr_ref[...]
    m_new = jnp.maximum(m_old, pm)
    resc = jnp.where(m_old > -jnp.inf, jnp.exp(m_old - m_new), 0.0)

    m_new_col = m_new[:, 0:1]
    mv = jnp.where(sel, jnp.broadcast_to(m_new_col, (_NCP, _T)), -jnp.inf)
    m_pp = jnp.max(mv, axis=0, keepdims=True)
    w = jnp.exp(m - m_pp)
    wm = maskf * w

    x_bf = x.astype(jnp.bfloat16)
    r = jax.lax.dot_general(wm, x, (((1,), (1,)), ((), ())),
                            precision=jax.lax.Precision.HIGHEST,
                            preferred_element_type=jnp.float32)
    se = jnp.sum(wm, axis=1, keepdims=True)

    x1 = jax.lax.dot_general(
        w1_ref[...].astype(jnp.bfloat16), x_bf,
        (((1,), (0,)), ((), ())), preferred_element_type=jnp.float32)
    x1 = x1 + b1_ref[:, 0:1]
    ra = jax.lax.dot_general(
        maskf.astype(jnp.bfloat16), x1.astype(jnp.bfloat16),
        (((1,), (1,)), ((), ())),
        preferred_element_type=jnp.float32)

    nw_ref[...] = nw_ref[...] * resc + r
    at_ref[...] = at_ref[...] + ra
    d_ref[...] = d_ref[...] * resc + se
    mr_ref[...] = m_new

    @pl.when(j == nblk - 1)
    def _fin():
        d = d_ref[...]
        d_safe = jnp.where(d > 0, d, 1.0)
        cls_feat = nw_ref[...] / d_safe
        cf2 = jax.lax.dot_general(
            cls_feat.astype(jnp.bfloat16), w2_ref[...].astype(jnp.bfloat16),
            (((1,), (1,)), ((), ())),
            preferred_element_type=jnp.float32) + b2_ref[...]
        cf2 = cf2.astype(jnp.bfloat16).astype(jnp.float32)
        gc = jax.lax.dot_general(at_ref[...], cf2, (((0,), (0,)), ((), ())),
                                 precision=jax.lax.Precision.HIGHEST,
                                 preferred_element_type=jnp.float32)
        gc = gc * (c ** -0.5)
        gmax = jnp.max(gc, axis=1, keepdims=True)
        e = jnp.exp(gc - gmax)
        out_ref[0] = e / jnp.sum(e, axis=1, keepdims=True)


def kernel(x, preds, W1, b1, W2, b2):
    b, c, h, w = x.shape
    hw = h * w
    nc = preds.shape[1]
    x2 = x.reshape(b, c, hw)
    p2 = preds.reshape(b, nc, hw)
    nblk = hw // _T
    m, k = _sc_route(p2)
    m3 = m.reshape(b * nblk, 1, _T)
    k3 = k.reshape(b * nblk, 1, _T)
    b1t = jnp.broadcast_to(b1[:, None], (c, 128))
    out = pl.pallas_call(
        _tc_body,
        grid=(b, nblk),
        in_specs=[
            pl.BlockSpec((1, c, _T), lambda i, j: (i, 0, j)),
            pl.BlockSpec((1, 1, _T), lambda i, j, n=nblk: (i * n + j, 0, 0)),
            pl.BlockSpec((1, 1, _T), lambda i, j, n=nblk: (i * n + j, 0, 0)),
            pl.BlockSpec((c, c), lambda i, j: (0, 0)),
            pl.BlockSpec((c, 128), lambda i, j: (0, 0)),
            pl.BlockSpec((c, c), lambda i, j: (0, 0)),
            pl.BlockSpec((1, c), lambda i, j: (0, 0)),
        ],
        out_specs=pl.BlockSpec((1, c, c), lambda i, j: (i, 0, 0)),
        out_shape=jax.ShapeDtypeStruct((b, c, c), jnp.float32),
        scratch_shapes=[pltpu.VMEM((_NCP, 128), jnp.float32)] * 4,
        compiler_params=pltpu.CompilerParams(
            dimension_semantics=("arbitrary", "arbitrary")),
    )(x2, m3, k3, W1, b1t, W2, b2.reshape(1, c))
    return out


# speedup vs baseline: 1.1051x; 1.1051x over previous
import jax
import jax.numpy as jnp
from jax import lax
from jax.experimental import pallas as pl
from jax.experimental.pallas import tpu as pltpu
from jax.experimental.pallas import tpu_sc as plsc

_NC = 19
_NCP = 24
_T = 8192

_NW = 32
_CH = 1536


def _sc_route(preds):
    b, nc, hw = preds.shape
    per_w = hw // _NW
    nch = per_w // _CH
    mesh = plsc.VectorSubcoreMesh(core_axis_name="c", subcore_axis_name="s")

    def body(preds_hbm, m_hbm, k_hbm, rows_v, m_buf, k_buf, sem):
        wid = lax.axis_index("s") * 2 + lax.axis_index("c")
        base = wid * per_w
        for bi in range(b):
            for ci in range(nch):
                start = base + ci * _CH
                cps = [
                    pltpu.async_copy(
                        preds_hbm.at[pl.ds((bi * nc + cls) * hw + start, _CH)],
                        rows_v.at[pl.ds(cls * _CH, _CH)], sem)
                    for cls in range(nc)
                ]
                for cp in cps:
                    cp.wait()

                def step(i, carry):
                    off = i * 16
                    mv = rows_v[pl.ds(off, 16)]
                    kv = jnp.zeros((16,), jnp.int32)
                    for cls in range(1, nc):
                        v = rows_v[pl.ds(cls * _CH + off, 16)]
                        gt = v > mv
                        mv = jnp.where(gt, v, mv)
                        kv = jnp.where(gt, cls, kv)
                    m_buf[pl.ds(off, 16)] = mv
                    k_buf[pl.ds(off, 16)] = kv
                    return carry

                lax.fori_loop(0, _CH // 16, step, 0)
                pltpu.sync_copy(m_buf, m_hbm.at[pl.ds(bi * hw + start, _CH)])
                pltpu.sync_copy(k_buf, k_hbm.at[pl.ds(bi * hw + start, _CH)])

    f = pl.kernel(
        body,
        mesh=mesh,
        out_type=[
            jax.ShapeDtypeStruct((b * hw,), jnp.float32),
            jax.ShapeDtypeStruct((b * hw,), jnp.int32),
        ],
        scratch_types=[
            pltpu.VMEM((nc * _CH,), jnp.float32),
            pltpu.VMEM((_CH,), jnp.float32),
            pltpu.VMEM((_CH,), jnp.int32),
            pltpu.SemaphoreType.DMA,
        ],
    )
    return f(preds.reshape(-1))


def _tc_body(x_ref, m_ref, k_ref, w1_ref, b1_ref, w2_ref, b2_ref, out_ref,
             nw_ref, at_ref, mr_ref, d_ref):
    j = pl.program_id(1)
    nblk = pl.num_programs(1)
    c = x_ref.shape[1]

    @pl.when(j == 0)
    def _init():
        nw_ref[...] = jnp.zeros_like(nw_ref)
        at_ref[...] = jnp.zeros_like(at_ref)
        d_ref[...] = jnp.zeros_like(d_ref)
        mr_ref[...] = jnp.full_like(mr_ref, -jnp.inf)

    x = x_ref[0]
    m = m_ref[0]
    k = k_ref[0]
    rows = jax.lax.broadcasted_iota(jnp.int32, (_NCP, _T), 0)
    sel = rows == k
    maskf = sel.astype(jnp.float32)

    mb = jnp.where(sel, jnp.broadcast_to(m, (_NCP, _T)), -jnp.inf)
    pm = jnp.max(mb, axis=1, keepdims=True)
    m_old = mr_ref[...]
    m_new = jnp.maximum(m_old, pm)
    resc = jnp.where(m_old > -jnp.inf, jnp.exp(m_old - m_new), 0.0)

    m_new_col = m_new[:, 0:1]
    mv = jnp.where(sel, jnp.broadcast_to(m_new_col, (_NCP, _T)), -jnp.inf)
    m_pp = jnp.max(mv, axis=0, keepdims=True)
    w = jnp.exp(m - m_pp)
    wm = maskf * w

    x_bf = x.astype(jnp.bfloat16)
    x_lo = (x - x_bf.astype(jnp.float32)).astype(jnp.bfloat16)
    wm_bf = wm.astype(jnp.bfloat16)
    wm_lo = (wm - wm_bf.astype(jnp.float32)).astype(jnp.bfloat16)
    dims = (((1,), (1,)), ((), ()))
    r = (jax.lax.dot_general(wm_bf, x_bf, dims,
                             preferred_element_type=jnp.float32)
         + jax.lax.dot_general(wm_bf, x_lo, dims,
                               preferred_element_type=jnp.float32)
         + jax.lax.dot_general(wm_lo, x_bf, dims,
                               preferred_element_type=jnp.float32))
    se = jnp.sum(wm, axis=1, keepdims=True)

    x1 = jax.lax.dot_general(
        w1_ref[...].astype(jnp.bfloat16), x_bf,
        (((1,), (0,)), ((), ())), preferred_element_type=jnp.float32)
    x1 = x1 + b1_ref[:, 0:1]
    ra = jax.lax.dot_general(
        maskf.astype(jnp.bfloat16), x1.astype(jnp.bfloat16),
        (((1,), (1,)), ((), ())),
        preferred_element_type=jnp.float32)

    nw_ref[...] = nw_ref[...] * resc + r
    at_ref[...] = at_ref[...] + ra
    d_ref[...] = d_ref[...] * resc + se
    mr_ref[...] = m_new

    @pl.when(j == nblk - 1)
    def _fin():
        d = d_ref[...]
        d_safe = jnp.where(d > 0, d, 1.0)
        cls_feat = nw_ref[...] / d_safe
        cf2 = jax.lax.dot_general(
            cls_feat.astype(jnp.bfloat16), w2_ref[...].astype(jnp.bfloat16),
            (((1,), (1,)), ((), ())),
            preferred_element_type=jnp.float32) + b2_ref[...]
        cf2 = cf2.astype(jnp.bfloat16).astype(jnp.float32)
        gc = jax.lax.dot_general(at_ref[...], cf2, (((0,), (0,)), ((), ())),
                                 precision=jax.lax.Precision.HIGHEST,
                                 preferred_element_type=jnp.float32)
        gc = gc * (c ** -0.5)
        gmax = jnp.max(gc, axis=1, keepdims=True)
        e = jnp.exp(gc - gmax)
        out_ref[0] = e / jnp.sum(e, axis=1, keepdims=True)


def kernel(x, preds, W1, b1, W2, b2):
    b, c, h, w = x.shape
    hw = h * w
    nc = preds.shape[1]
    x2 = x.reshape(b, c, hw)
    p2 = preds.reshape(b, nc, hw)
    nblk = hw // _T
    m, k = _sc_route(p2)
    m3 = m.reshape(b * nblk, 1, _T)
    k3 = k.reshape(b * nblk, 1, _T)
    b1t = jnp.broadcast_to(b1[:, None], (c, 128))
    out = pl.pallas_call(
        _tc_body,
        grid=(b, nblk),
        in_specs=[
            pl.BlockSpec((1, c, _T), lambda i, j: (i, 0, j)),
            pl.BlockSpec((1, 1, _T), lambda i, j, n=nblk: (i * n + j, 0, 0)),
            pl.BlockSpec((1, 1, _T), lambda i, j, n=nblk: (i * n + j, 0, 0)),
            pl.BlockSpec((c, c), lambda i, j: (0, 0)),
            pl.BlockSpec((c, 128), lambda i, j: (0, 0)),
            pl.BlockSpec((c, c), lambda i, j: (0, 0)),
            pl.BlockSpec((1, c), lambda i, j: (0, 0)),
        ],
        out_specs=pl.BlockSpec((1, c, c), lambda i, j: (i, 0, 0)),
        out_shape=jax.ShapeDtypeStruct((b, c, c), jnp.float32),
        scratch_shapes=[pltpu.VMEM((_NCP, 128), jnp.float32)] * 4,
        compiler_params=pltpu.CompilerParams(
            dimension_semantics=("arbitrary", "arbitrary")),
    )(x2, m3, k3, W1, b1t, W2, b2.reshape(1, c))
    return out
